# baseline (speedup 1.0000x reference)
import math

import jax
import jax.numpy as jnp
from jax.experimental import pallas as pl
from jax.experimental.pallas import tpu as pltpu

_COORD = 16
_MAXPOS = 512


def _fused_body(x_ref, ids_ref, idx_ref, wc_ref, bc_ref, wl_ref, bl_ref,
                lang_ref, smap_ref, lcol_ref, tvt_ref, cos_ref, ssin_ref,
                ov_ref, ot_ref, osv_ref, ost_ref):
    d = ov_ref.shape[-1]
    tm = x_ref.shape[0]
    v_lang = lang_ref.shape[0]
    cos = cos_ref[...]
    ssin = ssin_ref[...]

    def rope(h):
        return h * cos + pltpu.roll(h, shift=d // 2, axis=1) * ssin

    h = jnp.dot(x_ref[...], wc_ref[...], preferred_element_type=jnp.float32)
    h = jnp.maximum(h + bc_ref[...], 0.0)
    h = jnp.dot(h.astype(jnp.bfloat16), wl_ref[...],
                preferred_element_type=jnp.float32) + bl_ref[...]
    ov_ref[...] = rope(h)

    iota_l = jax.lax.broadcasted_iota(jnp.int32, (tm, v_lang), 1)
    onehot = (iota_l == ids_ref[...]).astype(jnp.bfloat16)
    ot_ref[...] = jnp.dot(onehot, lang_ref[...],
                          preferred_element_type=jnp.float32)

    spread = jnp.dot(idx_ref[...], smap_ref[...],
                     preferred_element_type=jnp.float32)
    mh = (spread == lcol_ref[...]).astype(jnp.bfloat16)
    evt = jnp.dot(mh, tvt_ref[...], preferred_element_type=jnp.float32)
    osv_ref[...] = rope(evt[:, :d])
    ost_ref[...] = rope(evt[:, d:])


def _pick_tile(n, period):
    for t in (4096, 2048, 1024, 512):
        if t % period == 0 and n % t == 0:
            return t
    return period


def kernel(conv_w, conv_b, lin_w, lin_b, lang_emb, spatial_table_v,
           spatial_table_t, spatial_slot_offsets, inv_freq_dup,
           pixel_values, input_ids, x_features, y_features):
    d = lang_emb.shape[1]
    bs, n_patches, k = pixel_values.shape
    seq = min(n_patches, _MAXPOS)
    b, s = input_ids.shape
    n = bs * seq
    assert b * s == n and s == seq, "fused row tiling requires matching seqs"

    tile = _pick_tile(n, seq)
    grid = n // tile

    xb = pixel_values[:, :seq, :].reshape(n, k)
    ids = input_ids.reshape(n, 1).astype(jnp.int32)

    def prep(f):
        f = f.astype(jnp.float32)
        return jnp.concatenate(
            [f[..., :3], jnp.clip(f[..., 3:], -_COORD, _COORD) + _COORD],
            axis=-1)

    idx = jnp.concatenate([prep(x_features), prep(y_features)], axis=-1)
    idx = idx.reshape(n, 16)

    vtot = spatial_table_v.shape[0]
    nslot = spatial_slot_offsets.shape[0]
    col = jnp.arange(vtot, dtype=jnp.int32)[None, :]
    offs = spatial_slot_offsets.astype(jnp.int32)[:, None]
    ends = jnp.concatenate(
        [offs[1:], jnp.full((1, 1), vtot, jnp.int32)], axis=0)
    smap = ((col >= offs) & (col < ends)).astype(jnp.float32)
    lcol = (col - (smap.astype(jnp.int32) * offs).sum(0, keepdims=True)
            ).astype(jnp.float32)
    tvt = jnp.concatenate([spatial_table_v, spatial_table_t],
                          axis=1).astype(jnp.bfloat16)

    pos = (jnp.arange(tile, dtype=jnp.int32) % seq).astype(jnp.float32)
    angle = pos[:, None] * inv_freq_dup.reshape(1, d)
    cos = jnp.cos(angle)
    sign = jnp.where(jnp.arange(d)[None, :] < d // 2, -1.0, 1.0)
    ssin = jnp.sin(angle) * sign

    wc = conv_w.reshape(d, k).T
    bc = conv_b.reshape(1, d)
    wl = lin_w.T.astype(jnp.bfloat16)
    bl = lin_b.reshape(1, d)
    lang = lang_emb.astype(jnp.bfloat16)

    row_spec = lambda w: pl.BlockSpec((tile, w), lambda i: (i, 0))
    res_spec = lambda h, w: pl.BlockSpec((h, w), lambda i: (0, 0))

    v_lang = lang_emb.shape[0]
    cost = pl.CostEstimate(
        flops=2 * n * (k * d + d * d + v_lang * d + 16 * vtot + vtot * 2 * d),
        transcendentals=0,
        bytes_accessed=4 * n * k + 4 * n * 17 + 16 * n * d + 2 * vtot * 3 * d)

    ov, ot, osv, ost = pl.pallas_call(
        _fused_body,
        out_shape=(jax.ShapeDtypeStruct((n, d), jnp.float32),) * 4,
        grid_spec=pltpu.PrefetchScalarGridSpec(
            num_scalar_prefetch=0,
            grid=(grid,),
            in_specs=[
                row_spec(k),
                row_spec(1),
                row_spec(16),
                res_spec(k, d),
                res_spec(1, d),
                res_spec(d, d),
                res_spec(1, d),
                res_spec(v_lang, d),
                res_spec(nslot, vtot),
                res_spec(1, vtot),
                res_spec(vtot, 2 * d),
                res_spec(tile, d),
                res_spec(tile, d),
            ],
            out_specs=(row_spec(d),) * 4,
        ),
        compiler_params=pltpu.CompilerParams(
            dimension_semantics=("parallel",)),
        cost_estimate=cost,
    )(xb, ids, idx, wc, bc, wl, bl, lang, smap, lcol, tvt, cos, ssin)

    return (ov.reshape(bs, seq, d), ot.reshape(b, s, d),
            osv.reshape(b, s, d), ost.reshape(b, s, d))


# speedup vs baseline: 1.5441x; 1.5441x over previous
import math

import jax
import jax.numpy as jnp
from jax.experimental import pallas as pl
from jax.experimental.pallas import tpu as pltpu

_COORD = 16
_MAXPOS = 512
_BB = 8


def _dot_t(a, b):
    return jax.lax.dot_general(a, b, (((0,), (0,)), ((), ())),
                               preferred_element_type=jnp.float32)


def _fused_body(xp_ref, ids_ref, xf_ref, yf_ref, wc_ref, bc_ref, wl_ref,
                bl_ref, lang_ref, smap_ref, lcol_ref, tvt_ref, cos_ref,
                ssin_ref, ov_ref, ot_ref, osv_ref, ost_ref):
    d = ov_ref.shape[-1]
    s = xp_ref.shape[-1]
    v_lang = lang_ref.shape[0]
    cos = cos_ref[...]
    ssin = ssin_ref[...]
    lcol = lcol_ref[...]

    def rope(h):
        return h * cos + pltpu.roll(h, shift=d // 2, axis=1) * ssin

    for j in range(_BB):
        rows = slice(j * s, (j + 1) * s)

        h = _dot_t(xp_ref[j], wc_ref[...])
        h = jnp.maximum(h + bc_ref[...], 0.0)
        h = jnp.dot(h, wl_ref[...],
                    preferred_element_type=jnp.float32) + bl_ref[...]
        ov_ref[rows, :] = rope(h)

        iota_v = jax.lax.broadcasted_iota(jnp.int32, (v_lang, s), 0)
        oh_t = (iota_v == ids_ref[j:j + 1, :]).astype(jnp.bfloat16)
        ot_ref[rows, :] = _dot_t(oh_t, lang_ref[...])

        iota_c = jax.lax.broadcasted_iota(jnp.int32, (8, s), 0)
        fx = xf_ref[j].astype(jnp.float32)
        fy = yf_ref[j].astype(jnp.float32)
        fx = jnp.where(iota_c < 3, fx,
                       jnp.clip(fx, -_COORD, _COORD) + _COORD)
        fy = jnp.where(iota_c < 3, fy,
                       jnp.clip(fy, -_COORD, _COORD) + _COORD)
        f16 = jnp.concatenate([fx, fy], axis=0)
        spread = _dot_t(f16, smap_ref[...])
        mh = (spread == lcol).astype(jnp.bfloat16)
        evt = jnp.dot(mh, tvt_ref[...],
                      preferred_element_type=jnp.float32)
        osv_ref[rows, :] = rope(evt[:, :d])
        ost_ref[rows, :] = rope(evt[:, d:])


def kernel(conv_w, conv_b, lin_w, lin_b, lang_emb, spatial_table_v,
           spatial_table_t, spatial_slot_offsets, inv_freq_dup,
           pixel_values, input_ids, x_features, y_features):
    d = lang_emb.shape[1]
    bs, n_patches, k = pixel_values.shape
    seq = min(n_patches, _MAXPOS)
    b, s = input_ids.shape
    n = bs * seq
    assert b == bs and s == seq and bs % _BB == 0, "fused batch tiling"
    grid = bs // _BB
    tile = _BB * seq

    xp = jnp.transpose(pixel_values, (0, 2, 1))[:, :, :seq]
    xf = jnp.transpose(x_features, (0, 2, 1))
    yf = jnp.transpose(y_features, (0, 2, 1))
    ids = input_ids.astype(jnp.int32)

    vtot = spatial_table_v.shape[0]
    nslot = spatial_slot_offsets.shape[0]
    col = jnp.arange(vtot, dtype=jnp.int32)[None, :]
    offs = spatial_slot_offsets.astype(jnp.int32)[:, None]
    ends = jnp.concatenate(
        [offs[1:], jnp.full((1, 1), vtot, jnp.int32)], axis=0)
    smap = ((col >= offs) & (col < ends)).astype(jnp.float32)
    lcol = (col - (smap.astype(jnp.int32) * offs).sum(0, keepdims=True)
            ).astype(jnp.float32)
    tvt = jnp.concatenate([spatial_table_v, spatial_table_t],
                          axis=1).astype(jnp.bfloat16)

    pos = jnp.arange(seq, dtype=jnp.float32)
    angle = pos[:, None] * inv_freq_dup.reshape(1, d)
    cos = jnp.cos(angle)
    sign = jnp.where(jnp.arange(d)[None, :] < d // 2, -1.0, 1.0)
    ssin = jnp.sin(angle) * sign

    wc = conv_w.reshape(d, k).T
    bc = conv_b.reshape(1, d)
    wl = lin_w.T
    bl = lin_b.reshape(1, d)
    lang = lang_emb.astype(jnp.bfloat16)
    v_lang = lang_emb.shape[0]

    blk3 = lambda c: pl.BlockSpec((_BB, c, seq), lambda i: (i, 0, 0))
    row_out = pl.BlockSpec((tile, d), lambda i: (i, 0))
    res_spec = lambda h, w: pl.BlockSpec((h, w), lambda i: (0, 0))

    cost = pl.CostEstimate(
        flops=2 * n * (k * d + d * d + v_lang * d + 16 * vtot
                       + vtot * 2 * d),
        transcendentals=0,
        bytes_accessed=4 * n * (k + 17) + 16 * n * d + 2 * vtot * 3 * d)

    ov, ot, osv, ost = pl.pallas_call(
        _fused_body,
        out_shape=(jax.ShapeDtypeStruct((n, d), jnp.float32),) * 4,
        grid_spec=pltpu.PrefetchScalarGridSpec(
            num_scalar_prefetch=0,
            grid=(grid,),
            in_specs=[
                blk3(k),
                pl.BlockSpec((_BB, seq), lambda i: (i, 0)),
                blk3(8),
                blk3(8),
                res_spec(k, d),
                res_spec(1, d),
                res_spec(d, d),
                res_spec(1, d),
                res_spec(v_lang, d),
                res_spec(nslot, vtot),
                res_spec(1, vtot),
                res_spec(vtot, 2 * d),
                res_spec(seq, d),
                res_spec(seq, d),
            ],
            out_specs=(row_out,) * 4,
        ),
        compiler_params=pltpu.CompilerParams(
            dimension_semantics=("parallel",)),
        cost_estimate=cost,
    )(xp, ids, xf, yf, wc, bc, wl, bl, lang, smap, lcol, tvt, cos, ssin)

    return (ov.reshape(bs, seq, d), ot.reshape(b, s, d),
            osv.reshape(b, s, d), ost.reshape(b, s, d))


# speedup vs baseline: 1.6087x; 1.0418x over previous
import math

import jax
import jax.numpy as jnp
from jax.experimental import pallas as pl
from jax.experimental.pallas import tpu as pltpu

_COORD = 16
_MAXPOS = 512
_BB = 16


def _dot_t(a, b):
    return jax.lax.dot_general(a, b, (((0,), (0,)), ((), ())),
                               preferred_element_type=jnp.float32)


def _fused_body(xp_ref, ids_ref, xf_ref, yf_ref, wc_ref, bc_ref, wl_ref,
                bl_ref, lang_ref, smap_ref, lcol_ref, tvt_ref, cos_ref,
                ssin_ref, ov_ref, ot_ref, osv_ref, ost_ref):
    d = ov_ref.shape[-1]
    s = xp_ref.shape[-1]
    v_lang = lang_ref.shape[0]
    cos = cos_ref[...]
    ssin = ssin_ref[...]
    lcol = lcol_ref[...]

    def rope(h):
        return h * cos + pltpu.roll(h, shift=d // 2, axis=1) * ssin

    for j in range(_BB):
        rows = slice(j * s, (j + 1) * s)

        h = _dot_t(xp_ref[j], wc_ref[...])
        h = jnp.maximum(h + bc_ref[...], 0.0)
        h = jnp.dot(h, wl_ref[...],
                    preferred_element_type=jnp.float32) + bl_ref[...]
        ov_ref[rows, :] = rope(h)

        iota_v = jax.lax.broadcasted_iota(jnp.int32, (v_lang, s), 0)
        oh_t = (iota_v == ids_ref[j:j + 1, :]).astype(jnp.bfloat16)
        ot_ref[rows, :] = _dot_t(oh_t, lang_ref[...])

        iota_c = jax.lax.broadcasted_iota(jnp.int32, (8, s), 0)
        fx = xf_ref[j].astype(jnp.float32)
        fy = yf_ref[j].astype(jnp.float32)
        fx = jnp.where(iota_c < 3, fx,
                       jnp.clip(fx, -_COORD, _COORD) + _COORD)
        fy = jnp.where(iota_c < 3, fy,
                       jnp.clip(fy, -_COORD, _COORD) + _COORD)
        f16 = jnp.concatenate([fx, fy], axis=0)
        spread = _dot_t(f16, smap_ref[...])
        mh = (spread == lcol).astype(jnp.bfloat16)
        evt = jnp.dot(mh, tvt_ref[...],
                      preferred_element_type=jnp.float32)
        osv_ref[rows, :] = rope(evt[:, :d])
        ost_ref[rows, :] = rope(evt[:, d:])


def kernel(conv_w, conv_b, lin_w, lin_b, lang_emb, spatial_table_v,
           spatial_table_t, spatial_slot_offsets, inv_freq_dup,
           pixel_values, input_ids, x_features, y_features):
    d = lang_emb.shape[1]
    bs, n_patches, k = pixel_values.shape
    seq = min(n_patches, _MAXPOS)
    b, s = input_ids.shape
    n = bs * seq
    assert b == bs and s == seq and bs % _BB == 0, "fused batch tiling"
    grid = bs // _BB
    tile = _BB * seq

    xp = jnp.transpose(pixel_values, (0, 2, 1))[:, :, :seq]
    xf = jnp.transpose(x_features, (0, 2, 1))
    yf = jnp.transpose(y_features, (0, 2, 1))
    ids = input_ids.astype(jnp.int32)

    vtot = spatial_table_v.shape[0]
    nslot = spatial_slot_offsets.shape[0]
    col = jnp.arange(vtot, dtype=jnp.int32)[None, :]
    offs = spatial_slot_offsets.astype(jnp.int32)[:, None]
    ends = jnp.concatenate(
        [offs[1:], jnp.full((1, 1), vtot, jnp.int32)], axis=0)
    smap = ((col >= offs) & (col < ends)).astype(jnp.float32)
    lcol = (col - (smap.astype(jnp.int32) * offs).sum(0, keepdims=True)
            ).astype(jnp.float32)
    tvt = jnp.concatenate([spatial_table_v, spatial_table_t],
                          axis=1).astype(jnp.bfloat16)

    pos = jnp.arange(seq, dtype=jnp.float32)
    angle = pos[:, None] * inv_freq_dup.reshape(1, d)
    cos = jnp.cos(angle)
    sign = jnp.where(jnp.arange(d)[None, :] < d // 2, -1.0, 1.0)
    ssin = jnp.sin(angle) * sign

    wc = conv_w.reshape(d, k).T
    bc = conv_b.reshape(1, d)
    wl = lin_w.T
    bl = lin_b.reshape(1, d)
    lang = lang_emb.astype(jnp.bfloat16)
    v_lang = lang_emb.shape[0]

    blk3 = lambda c: pl.BlockSpec((_BB, c, seq), lambda i: (i, 0, 0))
    row_out = pl.BlockSpec((tile, d), lambda i: (i, 0))
    res_spec = lambda h, w: pl.BlockSpec((h, w), lambda i: (0, 0))

    cost = pl.CostEstimate(
        flops=2 * n * (k * d + d * d + v_lang * d + 16 * vtot
                       + vtot * 2 * d),
        transcendentals=0,
        bytes_accessed=4 * n * (k + 17) + 16 * n * d + 2 * vtot * 3 * d)

    ov, ot, osv, ost = pl.pallas_call(
        _fused_body,
        out_shape=(jax.ShapeDtypeStruct((n, d), jnp.float32),) * 4,
        grid_spec=pltpu.PrefetchScalarGridSpec(
            num_scalar_prefetch=0,
            grid=(grid,),
            in_specs=[
                blk3(k),
                pl.BlockSpec((_BB, seq), lambda i: (i, 0)),
                blk3(8),
                blk3(8),
                res_spec(k, d),
                res_spec(1, d),
                res_spec(d, d),
                res_spec(1, d),
                res_spec(v_lang, d),
                res_spec(nslot, vtot),
                res_spec(1, vtot),
                res_spec(vtot, 2 * d),
                res_spec(seq, d),
                res_spec(seq, d),
            ],
            out_specs=(row_out,) * 4,
        ),
        compiler_params=pltpu.CompilerParams(
            dimension_semantics=("parallel",)),
        cost_estimate=cost,
    )(xp, ids, xf, yf, wc, bc, wl, bl, lang, smap, lcol, tvt, cos, ssin)

    return (ov.reshape(bs, seq, d), ot.reshape(b, s, d),
            osv.reshape(b, s, d), ost.reshape(b, s, d))
